# Initial kernel scaffold; baseline (speedup 1.0000x reference)
import jax
import jax.numpy as jnp
from jax.experimental import pallas as pl


def kernel(xyz1, points1, xyz2, points2, W1, b1, gamma1, beta1, rm1, rv1, W2, b2, gamma2, beta2, rm2, rv2):
    raise NotImplementedError("write your pallas kernel here")


# speedup vs baseline: 32.8072x; 32.8072x over previous
import functools

import jax
import jax.numpy as jnp
from jax.experimental import pallas as pl
from jax.experimental.pallas import tpu as pltpu

B, N1, N2 = 8, 1024, 4096
DIM1, DIM2, DOUT = 256, 128, 128
N2B = 1024
NJ = N2 // N2B


def _body(xyz1t_ref, points1_ref, xyz2p_ref, points2_ref,
          w1_ref, s1_ref, t1_ref, w2_ref, s2_ref, t2_ref,
          out_ref, feats1_ref):
    j = pl.program_id(1)

    @pl.when(j == 0)
    def _():
        f1 = jnp.dot(points1_ref[0], w1_ref[...],
                     preferred_element_type=jnp.float32)
        feats1_ref[...] = jnp.maximum(f1 * s1_ref[0] + t1_ref[0], 0.0)

    x1t = xyz1t_ref[0]
    x2 = xyz2p_ref[0]
    n1sq = (x1t[0:1, :] * x1t[0:1, :]
            + x1t[1:2, :] * x1t[1:2, :]
            + x1t[2:3, :] * x1t[2:3, :])
    n2sq = (x2[:, 0:1] * x2[:, 0:1]
            + x2[:, 1:2] * x2[:, 1:2]
            + x2[:, 2:3] * x2[:, 2:3])
    p = jnp.dot(x2, x1t, preferred_element_type=jnp.float32)
    d = -2.0 * p + n2sq + n1sq

    iota = jax.lax.broadcasted_iota(jnp.int32, (N2B, N1), 1).astype(jnp.float32)
    big_d = jnp.float32(1e30)
    big_i = jnp.float32(2.0 ** 30)
    mins, idxs = [], []
    for _k in range(3):
        mn = jnp.min(d, axis=1, keepdims=True)
        ik = jnp.min(jnp.where(d == mn, iota, big_i), axis=1, keepdims=True)
        mins.append(mn)
        idxs.append(ik)
        d = jnp.where(iota == ik, big_d, d)

    r = [1.0 / (m + 1e-8) for m in mins]
    norm = r[0] + r[1] + r[2]
    s = jnp.zeros((N2B, N1), jnp.float32)
    for k in range(3):
        s = s + jnp.where(iota == idxs[k], r[k] / norm, 0.0)

    interp = jnp.dot(s, feats1_ref[...], preferred_element_type=jnp.float32)
    f2 = jnp.dot(points2_ref[0], w2_ref[...],
                 preferred_element_type=jnp.float32)
    out_ref[0] = interp + jnp.maximum(f2 * s2_ref[0] + t2_ref[0], 0.0)


@functools.partial(jax.jit, static_argnames=())
def _run(xyz1t, points1, xyz2p, points2, w1, s1, t1, w2, s2, t2):
    return pl.pallas_call(
        _body,
        grid=(B, NJ),
        in_specs=[
            pl.BlockSpec((1, 8, N1), lambda b, j: (b, 0, 0)),
            pl.BlockSpec((1, N1, DIM1), lambda b, j: (b, 0, 0)),
            pl.BlockSpec((1, N2B, 8), lambda b, j: (b, j, 0)),
            pl.BlockSpec((1, N2B, DIM2), lambda b, j: (b, j, 0)),
            pl.BlockSpec((DIM1, DOUT), lambda b, j: (0, 0)),
            pl.BlockSpec((1, DOUT), lambda b, j: (0, 0)),
            pl.BlockSpec((1, DOUT), lambda b, j: (0, 0)),
            pl.BlockSpec((DIM2, DOUT), lambda b, j: (0, 0)),
            pl.BlockSpec((1, DOUT), lambda b, j: (0, 0)),
            pl.BlockSpec((1, DOUT), lambda b, j: (0, 0)),
        ],
        out_specs=pl.BlockSpec((1, N2B, DOUT), lambda b, j: (b, j, 0)),
        out_shape=jax.ShapeDtypeStruct((B, N2, DOUT), jnp.float32),
        scratch_shapes=[pltpu.VMEM((N1, DOUT), jnp.float32)],
    )(xyz1t, points1, xyz2p, points2, w1, s1, t1, w2, s2, t2)


def kernel(xyz1, points1, xyz2, points2, W1, b1, gamma1, beta1, rm1, rv1,
           W2, b2, gamma2, beta2, rm2, rv2):
    s1 = (gamma1 / jnp.sqrt(rv1 + 1e-5))[None, :]
    t1 = ((b1 - rm1) * s1[0] + beta1)[None, :]
    s2 = (gamma2 / jnp.sqrt(rv2 + 1e-5))[None, :]
    t2 = ((b2 - rm2) * s2[0] + beta2)[None, :]

    xyz2p = jnp.pad(xyz2, ((0, 0), (0, 0), (0, 5)))
    xyz1t = jnp.pad(xyz1, ((0, 0), (0, 0), (0, 5)))
    xyz1t = jnp.transpose(xyz1t, (0, 2, 1))

    return _run(xyz1t, points1, xyz2p, points2, W1.T, s1, t1, W2.T, s2, t2)
